# Initial kernel scaffold; baseline (speedup 1.0000x reference)
import jax
import jax.numpy as jnp
from jax.experimental import pallas as pl


def kernel(x, y, sw1, sb1, sg1, sbeta1, sw2, sb2, sg2, sbeta2, sw3, sb3, ww1, wb1, wg1, wbeta1, ww2, wb2, wg2, wbeta2, ww3, wb3, mw, mb, mg, mbeta, lw, lb, lg, lbeta):
    raise NotImplementedError("write your pallas kernel here")


# speedup vs baseline: 10.3674x; 10.3674x over previous
import functools

import jax
import jax.numpy as jnp
from jax import lax
from jax.experimental import pallas as pl
from jax.experimental.pallas import tpu as pltpu
from jax.experimental.pallas import tpu_sc as plsc

B = 4
N = 1024
M = 4096
K = 16
CF = 32
D2 = 80
TA = 128
TP = 256
DENS_C = float((2.0 * 3.141592653589793) ** -1.5)


def _topk_argmin(val, ii, k, nbig):
    ds, sels = [], []
    for _ in range(k):
        m = jnp.min(val, axis=1, keepdims=True)
        sel = jnp.min(jnp.where(val <= m, ii, nbig), axis=1, keepdims=True)
        ds.append(m)
        sels.append(sel)
        val = jnp.where(ii == sel, jnp.float32(3.0e38), val)
    return ds, sels


def _ka_body(qt_ref, qa_ref, pa_ref, wgt_ref, idx_ref, idxq_ref, dens_ref):
    b = pl.program_id(0)
    qt = qt_ref[0]
    qa = qa_ref[0]
    pa = pa_ref[0]
    qn = jnp.sum(qt * qt, axis=0)[:, None]
    pn = jnp.sum(pa * pa, axis=0)[None, :]
    qqn = jnp.sum(qa * qa, axis=0)[None, :]

    dot = lambda a, bb: lax.dot_general(a, bb, (((0,), (0,)), ((), ())),
                                        preferred_element_type=jnp.float32)
    d2p = jnp.maximum(qn + pn - 2.0 * dot(qt, pa), 0.0)
    iiN = lax.broadcasted_iota(jnp.int32, (TA, N), 1)
    ds, sels = _topk_argmin(d2p, iiN, K, jnp.int32(N))
    w = jnp.concatenate([1.0 / (m + 1e-8) for m in ds], axis=1)
    wgt_ref[0] = w / jnp.sum(w, axis=1, keepdims=True)
    idx_ref[0] = jnp.concatenate(sels, axis=1) + b * N

    d2q = jnp.maximum(qn + qqn - 2.0 * dot(qt, qa), 0.0)
    dens_ref[0, 0, :] = jnp.sum(jnp.exp(d2q * -0.5), axis=1) * (DENS_C / M)
    iiM = lax.broadcasted_iota(jnp.int32, (TA, M), 1)
    _, selq = _topk_argmin(d2q, iiM, K, jnp.int32(M))
    idxq_ref[0] = jnp.concatenate(selq, axis=1) + b * M


def _kernel_a(qpad, ppad):
    nt = M // TA
    return pl.pallas_call(
        _ka_body,
        grid=(B, nt),
        in_specs=[
            pl.BlockSpec((1, 8, TA), lambda b, t: (b, 0, t)),
            pl.BlockSpec((1, 8, M), lambda b, t: (b, 0, 0)),
            pl.BlockSpec((1, 8, N), lambda b, t: (b, 0, 0)),
        ],
        out_specs=[
            pl.BlockSpec((1, TA, K), lambda b, t: (b, t, 0)),
            pl.BlockSpec((1, TA, K), lambda b, t: (b, t, 0)),
            pl.BlockSpec((1, TA, K), lambda b, t: (b, t, 0)),
            pl.BlockSpec((1, 1, TA), lambda b, t: (b, 0, t)),
        ],
        out_shape=[
            jax.ShapeDtypeStruct((B, M, K), jnp.float32),
            jax.ShapeDtypeStruct((B, M, K), jnp.int32),
            jax.ShapeDtypeStruct((B, M, K), jnp.int32),
            jax.ShapeDtypeStruct((B, 1, M), jnp.float32),
        ],
    )(qpad, qpad, ppad)


def _kb_body(dens_ref, sw1_ref, sw2_ref, sw3_ref, sb3_ref, out_ref):
    inv = 1.0 / dens_ref[...]
    mm = lambda a, bb: lax.dot_general(a, bb, (((1,), (0,)), ((), ())),
                                       preferred_element_type=jnp.float32)

    def bn_relu(t):
        mu = jnp.mean(t, axis=1, keepdims=True)
        va = jnp.mean(t * t, axis=1, keepdims=True) - mu * mu
        return jnp.maximum((t - mu) * lax.rsqrt(va + 1e-5), 0.0)

    h = bn_relu(mm(sw1_ref[...], inv))
    h = bn_relu(mm(sw2_ref[...], h))
    sp = mm(sw3_ref[...], h) + sb3_ref[...]
    out_ref[...] = 1.0 / (1.0 + jnp.exp(-sp))


def _kernel_b(dens2, sw1, sw2, sw3, sb3):
    full = lambda s: pl.BlockSpec(s, lambda: tuple(0 for _ in s))
    return pl.pallas_call(
        _kb_body,
        in_specs=[full((1, B * M)), full((8, 1)), full((8, 8)),
                  full((1, 8)), full((1, 1))],
        out_specs=full((1, B * M)),
        out_shape=jax.ShapeDtypeStruct((1, B * M), jnp.float32),
    )(dens2, sw1, sw2, sw3, sb3)


def _sc_gather(table, idx, d):
    r = idx.shape[0]
    info = plsc.get_sparse_core_info()
    nw = info.num_cores * info.num_subcores
    per_w = r // nw
    chunk = 128
    mesh = plsc.VectorSubcoreMesh(core_axis_name="c", subcore_axis_name="s")

    @functools.partial(
        pl.kernel, mesh=mesh,
        compiler_params=pltpu.CompilerParams(use_tc_tiling_on_sc=False),
        out_type=jax.ShapeDtypeStruct((r, d), jnp.float32),
        scratch_types=[pltpu.VMEM((chunk,), jnp.int32),
                       pltpu.VMEM((chunk, d), jnp.float32),
                       pltpu.SemaphoreType.DMA],
    )
    def gk(table_hbm, idx_hbm, out_hbm, idx_v, rows_v, sem):
        wid = lax.axis_index("s") * info.num_cores + lax.axis_index("c")
        base = wid * per_w

        def body(i, carry):
            off = base + i * chunk
            pltpu.sync_copy(idx_hbm.at[pl.ds(off, chunk)], idx_v)
            pltpu.async_copy(table_hbm.at[idx_v], rows_v, sem).wait()
            pltpu.sync_copy(rows_v, out_hbm.at[pl.ds(off, chunk)])
            return carry

        lax.fori_loop(0, per_w // chunk, body, 0)

    return gk(table, idx)


def _d1_body(f_ref, w_ref, xi_ref):
    f3 = f_ref[0]
    w = w_ref[0]
    acc = f3[0] * w[:, 0:1]
    for k in range(1, K):
        acc = acc + f3[k] * w[:, k:k + 1]
    xi_ref[0] = acc


def _kernel_d1(feats4, wgt):
    nt = M // TP
    return pl.pallas_call(
        _d1_body,
        grid=(B, nt),
        in_specs=[
            pl.BlockSpec((1, K, TP, CF), lambda b, t: (b, 0, t, 0)),
            pl.BlockSpec((1, TP, K), lambda b, t: (b, t, 0)),
        ],
        out_specs=pl.BlockSpec((1, TP, CF), lambda b, t: (b, t, 0)),
        out_shape=jax.ShapeDtypeStruct((B, M, CF), jnp.float32),
    )(feats4, wgt)


def _first():
    return jnp.logical_and(pl.program_id(0) == 0, pl.program_id(1) == 0)


def _rowsum2(t):
    return (jnp.sum(t, axis=0, keepdims=True),
            jnp.sum(t * t, axis=0, keepdims=True))


def _mmT(x, w):
    return lax.dot_general(x, w, (((1,), (1,)), ((), ())),
                           preferred_element_type=jnp.float32)


def _bn_apply(t, st_ref, cnt, g_ref, be_ref):
    mu = st_ref[0:1, :] * (1.0 / cnt)
    va = st_ref[1:2, :] * (1.0 / cnt) - mu * mu
    return (t - mu) * lax.rsqrt(va + 1e-5) * g_ref[...] + be_ref[...]


def _p1_body(g_ref, q_ref, ww1_ref, mw_ref, t1_ref, st1_ref, stm_ref):
    g3 = g_ref[0]
    qx = q_ref[0]
    prel = (g3[:, :, 0:3] - qx[None, :, 0:3]).reshape(K * TP, 3)
    t1 = _mmT(prel, ww1_ref[...])
    t1_ref[0] = t1.reshape(K, TP, 8)
    u = (g3[:, :, 3:67] * g3[:, :, 67:68]).reshape(K * TP, 64)
    m1 = _mmT(u, mw_ref[...])

    @pl.when(_first())
    def _():
        st1_ref[...] = jnp.zeros_like(st1_ref)
        stm_ref[...] = jnp.zeros_like(stm_ref)

    s, ss = _rowsum2(t1)
    st1_ref[0:1, :] += s
    st1_ref[1:2, :] += ss
    s, ss = _rowsum2(m1)
    stm_ref[0:1, :] += s
    stm_ref[1:2, :] += ss


def _kernel_p1(tg4, qt, ww1, mw):
    nt = M // TP
    cst = lambda shp: pl.BlockSpec(shp, lambda b, t: tuple(0 for _ in shp))
    return pl.pallas_call(
        _p1_body,
        grid=(B, nt),
        in_specs=[
            pl.BlockSpec((1, K, TP, D2), lambda b, t: (b, 0, t, 0)),
            pl.BlockSpec((1, TP, 8), lambda b, t: (b, t, 0)),
            cst((8, 3)), cst((64, 64)),
        ],
        out_specs=[
            pl.BlockSpec((1, K, TP, 8), lambda b, t: (b, 0, t, 0)),
            cst((8, 8)), cst((8, 64)),
        ],
        out_shape=[
            jax.ShapeDtypeStruct((B, K, M, 8), jnp.float32),
            jax.ShapeDtypeStruct((8, 8), jnp.float32),
            jax.ShapeDtypeStruct((8, 64), jnp.float32),
        ],
    )(tg4, qt, ww1, mw)


CNT_KN = float(B * M * K)
CNT_N = float(B * M)


def _p2_body(t1_ref, st1_ref, wg1_ref, wb1_ref, ww2_ref, st2_ref):
    t1 = t1_ref[0].reshape(K * TP, 8)
    h1 = jnp.maximum(_bn_apply(t1, st1_ref, CNT_KN, wg1_ref, wb1_ref), 0.0)
    t2 = _mmT(h1, ww2_ref[...])

    @pl.when(_first())
    def _():
        st2_ref[...] = jnp.zeros_like(st2_ref)

    s, ss = _rowsum2(t2)
    st2_ref[0:1, :] += s
    st2_ref[1:2, :] += ss


def _kernel_p2(t1a, st1, wg1, wbeta1, ww2):
    nt = M // TP
    cst = lambda shp: pl.BlockSpec(shp, lambda b, t: tuple(0 for _ in shp))
    return pl.pallas_call(
        _p2_body,
        grid=(B, nt),
        in_specs=[
            pl.BlockSpec((1, K, TP, 8), lambda b, t: (b, 0, t, 0)),
            cst((8, 8)), cst((1, 8)), cst((1, 8)), cst((8, 8)),
        ],
        out_specs=cst((8, 8)),
        out_shape=jax.ShapeDtypeStruct((8, 8), jnp.float32),
    )(t1a, st1, wg1, wbeta1, ww2)


def _p3_body(g_ref, t1_ref, st1_ref, st2_ref, stm_ref,
             wg1_ref, wb1_ref, ww2_ref, wg2_ref, wb2_ref, ww3_ref, wb3_ref,
             mw_ref, mg_ref, mbe_ref, lwr_ref,
             z_ref, stz_ref):
    t1 = t1_ref[0].reshape(K * TP, 8)
    h1 = jnp.maximum(_bn_apply(t1, st1_ref, CNT_KN, wg1_ref, wb1_ref), 0.0)
    t2 = _mmT(h1, ww2_ref[...])
    h2 = jnp.maximum(_bn_apply(t2, st2_ref, CNT_KN, wg2_ref, wb2_ref), 0.0)
    w16 = (_mmT(h2, ww3_ref[...]) + wb3_ref[...]).reshape(K, TP, K)

    g3 = g_ref[0]
    u = (g3[:, :, 3:67] * g3[:, :, 67:68]).reshape(K * TP, 64)
    m1 = _mmT(u, mw_ref[...])
    mh = jnp.maximum(_bn_apply(m1, stm_ref, CNT_KN, mg_ref, mbe_ref),
                     0.0).reshape(K, TP, 64)

    o = w16[0][:, :, None] * mh[0][:, None, :]
    for k in range(1, K):
        o = o + w16[k][:, :, None] * mh[k][:, None, :]
    mmn = lambda a, w: lax.dot_general(a, w, (((1,), (0,)), ((), ())),
                                       preferred_element_type=jnp.float32)
    z = mmn(o[:, 0, :], lwr_ref[0])
    for kp in range(1, K):
        z = z + mmn(o[:, kp, :], lwr_ref[kp])
    z_ref[0] = z

    @pl.when(_first())
    def _():
        stz_ref[...] = jnp.zeros_like(stz_ref)

    s, ss = _rowsum2(z)
    stz_ref[0:1, :] += s
    stz_ref[1:2, :] += ss


def _kernel_p3(tg4, t1a, st1, st2, stm, wg1, wbeta1, ww2, wg2, wbeta2,
               ww3, wb3r, mw, mg, mbeta, lwr):
    nt = M // TP
    cst = lambda shp: pl.BlockSpec(shp, lambda b, t: tuple(0 for _ in shp))
    return pl.pallas_call(
        _p3_body,
        grid=(B, nt),
        in_specs=[
            pl.BlockSpec((1, K, TP, D2), lambda b, t: (b, 0, t, 0)),
            pl.BlockSpec((1, K, TP, 8), lambda b, t: (b, 0, t, 0)),
            cst((8, 8)), cst((8, 8)), cst((8, 64)),
            cst((1, 8)), cst((1, 8)), cst((8, 8)), cst((1, 8)), cst((1, 8)),
            cst((16, 8)), cst((1, 16)),
            cst((64, 64)), cst((1, 64)), cst((1, 64)), cst((16, 64, 64)),
        ],
        out_specs=[
            pl.BlockSpec((1, TP, 64), lambda b, t: (b, t, 0)),
            cst((8, 64)),
        ],
        out_shape=[
            jax.ShapeDtypeStruct((B, M, 64), jnp.float32),
            jax.ShapeDtypeStruct((8, 64), jnp.float32),
        ],
    )(tg4, t1a, st1, st2, stm, wg1, wbeta1, ww2, wg2, wbeta2, ww3, wb3r,
      mw, mg, mbeta, lwr)


def _p4_body(z_ref, stz_ref, lg_ref, lbe_ref, out_ref):
    z = jnp.maximum(_bn_apply(z_ref[0], stz_ref, CNT_N, lg_ref, lbe_ref), 0.0)
    out_ref[0] = z.T


def _kernel_p4(zpre, stz, lg, lbeta):
    nt = M // TP
    cst = lambda shp: pl.BlockSpec(shp, lambda b, t: tuple(0 for _ in shp))
    return pl.pallas_call(
        _p4_body,
        grid=(B, nt),
        in_specs=[
            pl.BlockSpec((1, TP, 64), lambda b, t: (b, t, 0)),
            cst((8, 64)), cst((1, 64)), cst((1, 64)),
        ],
        out_specs=pl.BlockSpec((1, 64, TP), lambda b, t: (b, 0, t)),
        out_shape=jax.ShapeDtypeStruct((B, 64, M), jnp.float32),
    )(zpre, stz, lg, lbeta)


def kernel(x, y, sw1, sb1, sg1, sbeta1, sw2, sb2, sg2, sbeta2, sw3, sb3,
           ww1, wb1, wg1, wbeta1, ww2, wb2, wg2, wbeta2, ww3, wb3,
           mw, mb, mg, mbeta, lw, lb, lg, lbeta):
    f32 = jnp.float32
    q = y[:, :3, :]
    qpad = jnp.concatenate([q, jnp.zeros((B, 5, M), f32)], axis=1)
    ppad = jnp.concatenate([x[:, :3, :], jnp.zeros((B, 5, N), f32)], axis=1)

    wgt, idx1, idxq, dens = _kernel_a(qpad, ppad)

    s_out = _kernel_b(dens.reshape(1, B * M), sw1, sw2, sw3,
                      sb3.reshape(1, 1))

    xf_t = jnp.transpose(x[:, 3:, :], (0, 2, 1)).reshape(B * N, CF)
    idx1_km = jnp.transpose(idx1, (0, 2, 1)).reshape(-1)
    feats = _sc_gather(xf_t, idx1_km, CF)
    xi = _kernel_d1(feats.reshape(B, K, M, CF), wgt)

    q_t = jnp.transpose(qpad, (0, 2, 1)).reshape(B * M, 8)
    yf_t = jnp.transpose(y[:, 3:, :], (0, 2, 1)).reshape(B * M, CF)
    tcat = jnp.concatenate(
        [q_t[:, 0:3], xi.reshape(B * M, CF), yf_t,
         s_out.reshape(B * M, 1), jnp.zeros((B * M, D2 - 68), f32)], axis=1)
    idxq_km = jnp.transpose(idxq, (0, 2, 1)).reshape(-1)
    tg = _sc_gather(tcat, idxq_km, D2).reshape(B, K, M, D2)

    qt3 = q_t.reshape(B, M, 8)
    t1a, st1, stm = _kernel_p1(tg, qt3, ww1, mw)
    g1 = wg1.reshape(1, 8)
    b1 = wbeta1.reshape(1, 8)
    st2 = _kernel_p2(t1a, st1, g1, b1, ww2)
    lwr = jnp.transpose(lw, (2, 1, 0))
    zpre, stz = _kernel_p3(
        tg, t1a, st1, st2, stm, g1, b1, ww2, wg2.reshape(1, 8),
        wbeta2.reshape(1, 8), ww3, wb3.reshape(1, 16), mw,
        mg.reshape(1, 64), mbeta.reshape(1, 64), lwr)
    z = _kernel_p4(zpre, stz, lg.reshape(1, 64), lbeta.reshape(1, 64))
    return jnp.concatenate([q, z], axis=1)


# speedup vs baseline: 11.4557x; 1.1050x over previous
import functools

import jax
import jax.numpy as jnp
from jax import lax
from jax.experimental import pallas as pl
from jax.experimental.pallas import tpu as pltpu
from jax.experimental.pallas import tpu_sc as plsc

B = 4
N = 1024
M = 4096
K = 16
CF = 32
D2 = 80
TA = 128
TP = 256
DENS_C = float((2.0 * 3.141592653589793) ** -1.5)


def _ka_body(qt_ref, qa_ref, pa_ref, xfr_ref, xi_ref, idxq_ref, dens_ref):
    b = pl.program_id(0)
    qt = qt_ref[0]
    qa = qa_ref[0]
    pa = pa_ref[0]
    qn = jnp.sum(qt * qt, axis=0)[:, None]
    pn = jnp.sum(pa * pa, axis=0)[None, :]
    qqn = jnp.sum(qa * qa, axis=0)[None, :]

    dot = lambda a, bb: lax.dot_general(a, bb, (((0,), (0,)), ((), ())),
                                        preferred_element_type=jnp.float32)
    d2p = jnp.maximum(qn + pn - 2.0 * dot(qt, pa), 0.0)
    val = d2p
    for _ in range(K):
        m = jnp.min(val, axis=1, keepdims=True)
        val = jnp.where(val == m, jnp.float32(3.0e38), val)
    w = jnp.where(d2p <= m, 1.0 / (d2p + 1e-8), 0.0)
    xi = lax.dot_general(w, xfr_ref[0], (((1,), (0,)), ((), ())),
                         preferred_element_type=jnp.float32)
    xi_ref[0] = xi / jnp.sum(w, axis=1, keepdims=True)

    d2q = jnp.maximum(qn + qqn - 2.0 * dot(qt, qa), 0.0)
    dens_ref[0, 0, :] = jnp.sum(jnp.exp(d2q * -0.5), axis=1) * (DENS_C / M)
    ii = lax.broadcasted_iota(jnp.int32, (TA, M), 1)
    sels = []
    for _ in range(K):
        m = jnp.min(d2q, axis=1, keepdims=True)
        sel = jnp.min(jnp.where(d2q <= m, ii, jnp.int32(M)),
                      axis=1, keepdims=True)
        sels.append(sel)
        d2q = jnp.where(ii == sel, jnp.float32(3.0e38), d2q)
    idxq_ref[0] = jnp.concatenate(sels, axis=1) + b * M


def _kernel_a(qpad, ppad, xfr):
    nt = M // TA
    return pl.pallas_call(
        _ka_body,
        grid=(B, nt),
        in_specs=[
            pl.BlockSpec((1, 8, TA), lambda b, t: (b, 0, t)),
            pl.BlockSpec((1, 8, M), lambda b, t: (b, 0, 0)),
            pl.BlockSpec((1, 8, N), lambda b, t: (b, 0, 0)),
            pl.BlockSpec((1, N, CF), lambda b, t: (b, 0, 0)),
        ],
        out_specs=[
            pl.BlockSpec((1, TA, CF), lambda b, t: (b, t, 0)),
            pl.BlockSpec((1, TA, K), lambda b, t: (b, t, 0)),
            pl.BlockSpec((1, 1, TA), lambda b, t: (b, 0, t)),
        ],
        out_shape=[
            jax.ShapeDtypeStruct((B, M, CF), jnp.float32),
            jax.ShapeDtypeStruct((B, M, K), jnp.int32),
            jax.ShapeDtypeStruct((B, 1, M), jnp.float32),
        ],
    )(qpad, qpad, ppad, xfr)


def _kb_body(dens_ref, sw1_ref, sw2_ref, sw3_ref, sb3_ref, out_ref):
    inv = 1.0 / dens_ref[...]
    mm = lambda a, bb: lax.dot_general(a, bb, (((1,), (0,)), ((), ())),
                                       preferred_element_type=jnp.float32)

    def bn_relu(t):
        mu = jnp.mean(t, axis=1, keepdims=True)
        va = jnp.mean(t * t, axis=1, keepdims=True) - mu * mu
        return jnp.maximum((t - mu) * lax.rsqrt(va + 1e-5), 0.0)

    h = bn_relu(mm(sw1_ref[...], inv))
    h = bn_relu(mm(sw2_ref[...], h))
    sp = mm(sw3_ref[...], h) + sb3_ref[...]
    out_ref[...] = 1.0 / (1.0 + jnp.exp(-sp))


def _kernel_b(dens2, sw1, sw2, sw3, sb3):
    full = lambda s: pl.BlockSpec(s, lambda: tuple(0 for _ in s))
    return pl.pallas_call(
        _kb_body,
        in_specs=[full((1, B * M)), full((8, 1)), full((8, 8)),
                  full((1, 8)), full((1, 1))],
        out_specs=full((1, B * M)),
        out_shape=jax.ShapeDtypeStruct((1, B * M), jnp.float32),
    )(dens2, sw1, sw2, sw3, sb3)


def _sc_gather(table, idx, d):
    r = idx.shape[0]
    info = plsc.get_sparse_core_info()
    nw = info.num_cores * info.num_subcores
    per_w = r // nw
    chunk = 128
    mesh = plsc.VectorSubcoreMesh(core_axis_name="c", subcore_axis_name="s")

    @functools.partial(
        pl.kernel, mesh=mesh,
        compiler_params=pltpu.CompilerParams(use_tc_tiling_on_sc=False),
        out_type=jax.ShapeDtypeStruct((r, d), jnp.float32),
        scratch_types=[pltpu.VMEM((chunk,), jnp.int32),
                       pltpu.VMEM((chunk, d), jnp.float32),
                       pltpu.SemaphoreType.DMA],
    )
    def gk(table_hbm, idx_hbm, out_hbm, idx_v, rows_v, sem):
        wid = lax.axis_index("s") * info.num_cores + lax.axis_index("c")
        base = wid * per_w

        def body(i, carry):
            off = base + i * chunk
            pltpu.sync_copy(idx_hbm.at[pl.ds(off, chunk)], idx_v)
            pltpu.async_copy(table_hbm.at[idx_v], rows_v, sem).wait()
            pltpu.sync_copy(rows_v, out_hbm.at[pl.ds(off, chunk)])
            return carry

        lax.fori_loop(0, per_w // chunk, body, 0)

    return gk(table, idx)


def _first():
    return jnp.logical_and(pl.program_id(0) == 0, pl.program_id(1) == 0)


def _rowsum2(t):
    return (jnp.sum(t, axis=0, keepdims=True),
            jnp.sum(t * t, axis=0, keepdims=True))


def _mmT(x, w):
    return lax.dot_general(x, w, (((1,), (1,)), ((), ())),
                           preferred_element_type=jnp.float32)


def _bn_apply(t, st_ref, cnt, g_ref, be_ref):
    mu = st_ref[0:1, :] * (1.0 / cnt)
    va = st_ref[1:2, :] * (1.0 / cnt) - mu * mu
    return (t - mu) * lax.rsqrt(va + 1e-5) * g_ref[...] + be_ref[...]


def _p1_body(g_ref, q_ref, ww1_ref, mw_ref, t1_ref, st1_ref, stm_ref):
    g3 = g_ref[0]
    qx = q_ref[0]
    prel = (g3[:, :, 0:3] - qx[None, :, 0:3]).reshape(K * TP, 3)
    t1 = _mmT(prel, ww1_ref[...])
    t1_ref[0] = t1.reshape(K, TP, 8)
    u = (g3[:, :, 3:67] * g3[:, :, 67:68]).reshape(K * TP, 64)
    m1 = _mmT(u, mw_ref[...])

    @pl.when(_first())
    def _():
        st1_ref[...] = jnp.zeros_like(st1_ref)
        stm_ref[...] = jnp.zeros_like(stm_ref)

    s, ss = _rowsum2(t1)
    st1_ref[0:1, :] += s
    st1_ref[1:2, :] += ss
    s, ss = _rowsum2(m1)
    stm_ref[0:1, :] += s
    stm_ref[1:2, :] += ss


def _kernel_p1(tg4, qt, ww1, mw):
    nt = M // TP
    cst = lambda shp: pl.BlockSpec(shp, lambda b, t: tuple(0 for _ in shp))
    return pl.pallas_call(
        _p1_body,
        grid=(B, nt),
        in_specs=[
            pl.BlockSpec((1, K, TP, D2), lambda b, t: (b, 0, t, 0)),
            pl.BlockSpec((1, TP, 8), lambda b, t: (b, t, 0)),
            cst((8, 3)), cst((64, 64)),
        ],
        out_specs=[
            pl.BlockSpec((1, K, TP, 8), lambda b, t: (b, 0, t, 0)),
            cst((8, 8)), cst((8, 64)),
        ],
        out_shape=[
            jax.ShapeDtypeStruct((B, K, M, 8), jnp.float32),
            jax.ShapeDtypeStruct((8, 8), jnp.float32),
            jax.ShapeDtypeStruct((8, 64), jnp.float32),
        ],
    )(tg4, qt, ww1, mw)


CNT_KN = float(B * M * K)
CNT_N = float(B * M)


def _p2_body(t1_ref, st1_ref, wg1_ref, wb1_ref, ww2_ref, st2_ref):
    t1 = t1_ref[0].reshape(K * TP, 8)
    h1 = jnp.maximum(_bn_apply(t1, st1_ref, CNT_KN, wg1_ref, wb1_ref), 0.0)
    t2 = _mmT(h1, ww2_ref[...])

    @pl.when(_first())
    def _():
        st2_ref[...] = jnp.zeros_like(st2_ref)

    s, ss = _rowsum2(t2)
    st2_ref[0:1, :] += s
    st2_ref[1:2, :] += ss


def _kernel_p2(t1a, st1, wg1, wbeta1, ww2):
    nt = M // TP
    cst = lambda shp: pl.BlockSpec(shp, lambda b, t: tuple(0 for _ in shp))
    return pl.pallas_call(
        _p2_body,
        grid=(B, nt),
        in_specs=[
            pl.BlockSpec((1, K, TP, 8), lambda b, t: (b, 0, t, 0)),
            cst((8, 8)), cst((1, 8)), cst((1, 8)), cst((8, 8)),
        ],
        out_specs=cst((8, 8)),
        out_shape=jax.ShapeDtypeStruct((8, 8), jnp.float32),
    )(t1a, st1, wg1, wbeta1, ww2)


def _p3_body(g_ref, t1_ref, st1_ref, st2_ref, stm_ref,
             wg1_ref, wb1_ref, ww2_ref, wg2_ref, wb2_ref, ww3_ref, wb3_ref,
             mw_ref, mg_ref, mbe_ref, lwr_ref,
             z_ref, stz_ref):
    t1 = t1_ref[0].reshape(K * TP, 8)
    h1 = jnp.maximum(_bn_apply(t1, st1_ref, CNT_KN, wg1_ref, wb1_ref), 0.0)
    t2 = _mmT(h1, ww2_ref[...])
    h2 = jnp.maximum(_bn_apply(t2, st2_ref, CNT_KN, wg2_ref, wb2_ref), 0.0)
    w16 = (_mmT(h2, ww3_ref[...]) + wb3_ref[...]).reshape(K, TP, K)

    g3 = g_ref[0]
    u = (g3[:, :, 3:67] * g3[:, :, 67:68]).reshape(K * TP, 64)
    m1 = _mmT(u, mw_ref[...])
    mh = jnp.maximum(_bn_apply(m1, stm_ref, CNT_KN, mg_ref, mbe_ref),
                     0.0).reshape(K, TP, 64)

    o = w16[0][:, :, None] * mh[0][:, None, :]
    for k in range(1, K):
        o = o + w16[k][:, :, None] * mh[k][:, None, :]
    mmn = lambda a, w: lax.dot_general(a, w, (((1,), (0,)), ((), ())),
                                       preferred_element_type=jnp.float32)
    z = mmn(o[:, 0, :], lwr_ref[0])
    for kp in range(1, K):
        z = z + mmn(o[:, kp, :], lwr_ref[kp])
    z_ref[0] = z

    @pl.when(_first())
    def _():
        stz_ref[...] = jnp.zeros_like(stz_ref)

    s, ss = _rowsum2(z)
    stz_ref[0:1, :] += s
    stz_ref[1:2, :] += ss


def _kernel_p3(tg4, t1a, st1, st2, stm, wg1, wbeta1, ww2, wg2, wbeta2,
               ww3, wb3r, mw, mg, mbeta, lwr):
    nt = M // TP
    cst = lambda shp: pl.BlockSpec(shp, lambda b, t: tuple(0 for _ in shp))
    return pl.pallas_call(
        _p3_body,
        grid=(B, nt),
        in_specs=[
            pl.BlockSpec((1, K, TP, D2), lambda b, t: (b, 0, t, 0)),
            pl.BlockSpec((1, K, TP, 8), lambda b, t: (b, 0, t, 0)),
            cst((8, 8)), cst((8, 8)), cst((8, 64)),
            cst((1, 8)), cst((1, 8)), cst((8, 8)), cst((1, 8)), cst((1, 8)),
            cst((16, 8)), cst((1, 16)),
            cst((64, 64)), cst((1, 64)), cst((1, 64)), cst((16, 64, 64)),
        ],
        out_specs=[
            pl.BlockSpec((1, TP, 64), lambda b, t: (b, t, 0)),
            cst((8, 64)),
        ],
        out_shape=[
            jax.ShapeDtypeStruct((B, M, 64), jnp.float32),
            jax.ShapeDtypeStruct((8, 64), jnp.float32),
        ],
    )(tg4, t1a, st1, st2, stm, wg1, wbeta1, ww2, wg2, wbeta2, ww3, wb3r,
      mw, mg, mbeta, lwr)


def _p4_body(z_ref, stz_ref, lg_ref, lbe_ref, out_ref):
    z = jnp.maximum(_bn_apply(z_ref[0], stz_ref, CNT_N, lg_ref, lbe_ref), 0.0)
    out_ref[0] = z.T


def _kernel_p4(zpre, stz, lg, lbeta):
    nt = M // TP
    cst = lambda shp: pl.BlockSpec(shp, lambda b, t: tuple(0 for _ in shp))
    return pl.pallas_call(
        _p4_body,
        grid=(B, nt),
        in_specs=[
            pl.BlockSpec((1, TP, 64), lambda b, t: (b, t, 0)),
            cst((8, 64)), cst((1, 64)), cst((1, 64)),
        ],
        out_specs=pl.BlockSpec((1, 64, TP), lambda b, t: (b, 0, t)),
        out_shape=jax.ShapeDtypeStruct((B, 64, M), jnp.float32),
    )(zpre, stz, lg, lbeta)


def kernel(x, y, sw1, sb1, sg1, sbeta1, sw2, sb2, sg2, sbeta2, sw3, sb3,
           ww1, wb1, wg1, wbeta1, ww2, wb2, wg2, wbeta2, ww3, wb3,
           mw, mb, mg, mbeta, lw, lb, lg, lbeta):
    f32 = jnp.float32
    q = y[:, :3, :]
    qpad = jnp.concatenate([q, jnp.zeros((B, 5, M), f32)], axis=1)
    ppad = jnp.concatenate([x[:, :3, :], jnp.zeros((B, 5, N), f32)], axis=1)

    xfr = jnp.transpose(x[:, 3:, :], (0, 2, 1))
    xi, idxq, dens = _kernel_a(qpad, ppad, xfr)

    s_out = _kernel_b(dens.reshape(1, B * M), sw1, sw2, sw3,
                      sb3.reshape(1, 1))

    q_t = jnp.transpose(qpad, (0, 2, 1)).reshape(B * M, 8)
    yf_t = jnp.transpose(y[:, 3:, :], (0, 2, 1)).reshape(B * M, CF)
    tcat = jnp.concatenate(
        [q_t[:, 0:3], xi.reshape(B * M, CF), yf_t,
         s_out.reshape(B * M, 1), jnp.zeros((B * M, D2 - 68), f32)], axis=1)
    idxq_km = jnp.transpose(idxq, (0, 2, 1)).reshape(-1)
    tg = _sc_gather(tcat, idxq_km, D2).reshape(B, K, M, D2)

    qt3 = q_t.reshape(B, M, 8)
    t1a, st1, stm = _kernel_p1(tg, qt3, ww1, mw)
    g1 = wg1.reshape(1, 8)
    b1 = wbeta1.reshape(1, 8)
    st2 = _kernel_p2(t1a, st1, g1, b1, ww2)
    lwr = jnp.transpose(lw, (2, 1, 0))
    zpre, stz = _kernel_p3(
        tg, t1a, st1, st2, stm, g1, b1, ww2, wg2.reshape(1, 8),
        wbeta2.reshape(1, 8), ww3, wb3.reshape(1, 16), mw,
        mg.reshape(1, 64), mbeta.reshape(1, 64), lwr)
    z = _kernel_p4(zpre, stz, lg.reshape(1, 64), lbeta.reshape(1, 64))
    return jnp.concatenate([q, z], axis=1)
